# baseline (speedup 1.0000x reference)
import functools

import jax
import jax.numpy as jnp
from jax import lax
from jax.experimental import pallas as pl
from jax.experimental.pallas import tpu as pltpu
from jax.experimental.pallas import tpu_sc as plsc

D = 64
NI = 4096
NJ = 200
NW = 32
C = 128
CP = C + 1

_mesh = plsc.VectorSubcoreMesh(core_axis_name="c", subcore_axis_name="s")

_TCB = 4096
_NE = 1_000_000
_HALF = _NE // 2
_SH = _HALF // _TCB
_A = _SH * _TCB


def _transpose_blk(wlo_ref, whi_ref, out_ref):
    out_ref[:, 0:D] = wlo_ref[...].T
    out_ref[:, D:2 * D] = whi_ref[...].T


_tc_transpose = pl.pallas_call(
    _transpose_blk,
    grid=(pl.cdiv(_HALF, _TCB),),
    in_specs=[
        pl.BlockSpec((D, _TCB), lambda i: (0, i)),
        pl.BlockSpec((D, _TCB), lambda i: (0, i + _SH)),
    ],
    out_specs=pl.BlockSpec((_TCB, 2 * D), lambda i: (i, 0)),
    out_shape=jax.ShapeDtypeStruct((_HALF, 2 * D), jnp.float32),
    compiler_params=pltpu.CompilerParams(
        dimension_semantics=("arbitrary",)),
)


def _packed_table(weight):
    wv = weight.T
    main = _tc_transpose(wv, wv)
    tail = wv[:, _A + _HALF:].T
    packed = lax.dynamic_update_slice(main, tail, (_A, 0))
    return packed.reshape(_NE, D)


def _remap(idx):
    i2 = idx * 2
    return jnp.where(
        idx < _A, i2,
        jnp.where(idx < 2 * _A + (_HALF - _A), i2 - (2 * _A - 1),
                  i2 - _NE))


@functools.partial(
    pl.kernel,
    mesh=_mesh,
    compiler_params=pltpu.CompilerParams(
        use_tc_tiling_on_sc=False, needs_layout_passes=False),
    out_type=jax.ShapeDtypeStruct((NJ, D // 8, NW, 8, C), jnp.float32),
    scratch_types=[
        pltpu.VMEM((NJ, C), jnp.int32),
        pltpu.VMEM((NJ, C), jnp.int32),
        pltpu.VMEM((C, D), jnp.float32),
        pltpu.VMEM((C, D), jnp.float32),
        pltpu.VMEM((C, D), jnp.float32),
        pltpu.VMEM((C, D), jnp.float32),
        pltpu.VMEM((D // 8, 8, CP), jnp.float32),
        pltpu.VMEM((D // 8, 8, CP), jnp.float32),
        pltpu.SemaphoreType.DMA,
        pltpu.SemaphoreType.DMA,
        pltpu.SemaphoreType.DMA,
        pltpu.SemaphoreType.DMA,
        pltpu.SemaphoreType.DMA,
        pltpu.SemaphoreType.DMA,
    ],
)
def _emb_add(x_hbm, c_hbm, w_hbm, out_hbm, ix_v, ic_v,
             gx0, gx1, gc0, gc1, tb0, tb1,
             sgx0, sgx1, sgc0, sgc1, ss0, ss1):
    wid = lax.axis_index("s") * 2 + lax.axis_index("c")
    gx = (gx0, gx1)
    gc = (gc0, gc1)
    tb = (tb0, tb1)
    sgx = (sgx0, sgx1)
    sgc = (sgc0, sgc1)
    ss = (ss0, ss1)
    iota = lax.iota(jnp.int32, 16)
    dr_idx = lax.bitwise_and(iota, jnp.full((16,), 7, jnp.int32))
    tr_base = lax.shift_right_logical(iota, jnp.full((16,), 3, jnp.int32))

    pltpu.sync_copy(x_hbm.at[wid], ix_v)
    pltpu.sync_copy(c_hbm.at[wid], ic_v)

    for b in range(2):
        pltpu.async_copy(w_hbm.at[ix_v.at[b]], gx[b], sgx[b])
        pltpu.async_copy(w_hbm.at[ic_v.at[b]], gc[b], sgc[b])

    def pair(k, carry):
        for b in range(2):
            j = 2 * k + b
            pltpu.make_async_copy(w_hbm.at[ix_v.at[j]], gx[b], sgx[b]).wait()
            pltpu.make_async_copy(w_hbm.at[ic_v.at[j]], gc[b], sgc[b]).wait()

            @pl.when(k > 0)
            def _():
                pltpu.make_async_copy(
                    tb[b].at[:, :, pl.ds(0, C)],
                    out_hbm.at[0, :, 0], ss[b]).wait()

            @plsc.parallel_loop(0, C, unroll=2)
            def _(i):
                coli = jnp.full((16,), i, jnp.int32)
                for d0 in range(D // 16):
                    sl = pl.ds(d0 * 16, 16)
                    v = gx[b][i, sl] + gc[b][i, sl]
                    plsc.store_scatter(
                        tb[b], [tr_base + (2 * d0), dr_idx, coli], v)

            @pl.when(j + 2 < NJ)
            def _():
                pltpu.async_copy(w_hbm.at[ix_v.at[j + 2]], gx[b], sgx[b])
                pltpu.async_copy(w_hbm.at[ic_v.at[j + 2]], gc[b], sgc[b])

            pltpu.async_copy(
                tb[b].at[:, :, pl.ds(0, C)],
                out_hbm.at[j, :, wid], ss[b])
        return carry

    lax.fori_loop(0, NJ // 2, pair, 0, unroll=False)

    for b in range(2):
        pltpu.make_async_copy(
            tb[b].at[:, :, pl.ds(0, C)],
            out_hbm.at[0, :, 0], ss[b]).wait()


def kernel(x, constant, weight):
    xt = _remap(x.astype(jnp.int32)).T.reshape(NJ, NW, C).transpose(1, 0, 2)
    ct = _remap(constant.astype(jnp.int32)).T.reshape(NJ, NW, C).transpose(1, 0, 2)
    res = _emb_add(xt, ct, _packed_table(weight))
    return res.transpose(2, 4, 0, 1, 3).reshape(NI, NJ, D)


# speedup vs baseline: 1.0699x; 1.0699x over previous
import functools

import jax
import jax.numpy as jnp
from jax import lax
from jax.experimental import pallas as pl
from jax.experimental.pallas import tpu as pltpu
from jax.experimental.pallas import tpu_sc as plsc

D = 64
NI = 4096
NJ = 200
NW = 32
C = 128
CP = C + 1

_mesh = plsc.VectorSubcoreMesh(core_axis_name="c", subcore_axis_name="s")

_TCB = 8192
_NE = 1_000_000
_HALF = _NE // 2
_SH = _HALF // _TCB
_A = _SH * _TCB


def _transpose_blk(wlo_ref, whi_ref, out_ref):
    out_ref[:, 0:D] = wlo_ref[...].T
    out_ref[:, D:2 * D] = whi_ref[...].T


_tc_transpose = pl.pallas_call(
    _transpose_blk,
    grid=(pl.cdiv(_HALF, _TCB),),
    in_specs=[
        pl.BlockSpec((D, _TCB), lambda i: (0, i)),
        pl.BlockSpec((D, _TCB), lambda i: (0, i + _SH)),
    ],
    out_specs=pl.BlockSpec((_TCB, 2 * D), lambda i: (i, 0)),
    out_shape=jax.ShapeDtypeStruct((_HALF, 2 * D), jnp.float32),
    compiler_params=pltpu.CompilerParams(
        dimension_semantics=("arbitrary",)),
)


def _packed_table(weight):
    wv = weight.T
    main = _tc_transpose(wv, wv)
    tail = wv[:, _A + _HALF:].T
    packed = lax.dynamic_update_slice(main, tail, (_A, 0))
    return packed.reshape(_NE, D)


def _remap(idx):
    i2 = idx * 2
    return jnp.where(
        idx < _A, i2,
        jnp.where(idx < 2 * _A + (_HALF - _A), i2 - (2 * _A - 1),
                  i2 - _NE))


@functools.partial(
    pl.kernel,
    mesh=_mesh,
    compiler_params=pltpu.CompilerParams(
        use_tc_tiling_on_sc=False, needs_layout_passes=False),
    out_type=jax.ShapeDtypeStruct((NJ, D // 8, NW, 8, C), jnp.float32),
    scratch_types=[
        pltpu.VMEM((NJ, C), jnp.int32),
        pltpu.VMEM((NJ, C), jnp.int32),
        pltpu.VMEM((C, D), jnp.float32),
        pltpu.VMEM((C, D), jnp.float32),
        pltpu.VMEM((C, D), jnp.float32),
        pltpu.VMEM((C, D), jnp.float32),
        pltpu.VMEM((D // 8, 8, CP), jnp.float32),
        pltpu.VMEM((D // 8, 8, CP), jnp.float32),
        pltpu.SemaphoreType.DMA,
        pltpu.SemaphoreType.DMA,
        pltpu.SemaphoreType.DMA,
        pltpu.SemaphoreType.DMA,
        pltpu.SemaphoreType.DMA,
        pltpu.SemaphoreType.DMA,
    ],
)
def _emb_add(x_hbm, c_hbm, w_hbm, out_hbm, ix_v, ic_v,
             gx0, gx1, gc0, gc1, tb0, tb1,
             sgx0, sgx1, sgc0, sgc1, ss0, ss1):
    wid = lax.axis_index("s") * 2 + lax.axis_index("c")
    gx = (gx0, gx1)
    gc = (gc0, gc1)
    tb = (tb0, tb1)
    sgx = (sgx0, sgx1)
    sgc = (sgc0, sgc1)
    ss = (ss0, ss1)
    iota = lax.iota(jnp.int32, 16)
    dr_idx = lax.bitwise_and(iota, jnp.full((16,), 7, jnp.int32))
    tr_base = lax.shift_right_logical(iota, jnp.full((16,), 3, jnp.int32))

    pltpu.sync_copy(x_hbm.at[wid], ix_v)
    pltpu.sync_copy(c_hbm.at[wid], ic_v)

    for b in range(2):
        pltpu.async_copy(w_hbm.at[ix_v.at[b]], gx[b], sgx[b])
        pltpu.async_copy(w_hbm.at[ic_v.at[b]], gc[b], sgc[b])

    def pair(k, carry):
        for b in range(2):
            j = 2 * k + b
            pltpu.make_async_copy(w_hbm.at[ix_v.at[j]], gx[b], sgx[b]).wait()
            pltpu.make_async_copy(w_hbm.at[ic_v.at[j]], gc[b], sgc[b]).wait()

            @pl.when(k > 0)
            def _():
                pltpu.make_async_copy(
                    tb[b].at[:, :, pl.ds(0, C)],
                    out_hbm.at[0, :, 0], ss[b]).wait()

            @plsc.parallel_loop(0, C, unroll=4)
            def _(i):
                coli = jnp.full((16,), i, jnp.int32)
                for d0 in range(D // 16):
                    sl = pl.ds(d0 * 16, 16)
                    v = gx[b][i, sl] + gc[b][i, sl]
                    plsc.store_scatter(
                        tb[b], [tr_base + (2 * d0), dr_idx, coli], v)

            @pl.when(j + 2 < NJ)
            def _():
                pltpu.async_copy(w_hbm.at[ix_v.at[j + 2]], gx[b], sgx[b])
                pltpu.async_copy(w_hbm.at[ic_v.at[j + 2]], gc[b], sgc[b])

            pltpu.async_copy(
                tb[b].at[:, :, pl.ds(0, C)],
                out_hbm.at[j, :, wid], ss[b])
        return carry

    lax.fori_loop(0, NJ // 2, pair, 0, unroll=False)

    for b in range(2):
        pltpu.make_async_copy(
            tb[b].at[:, :, pl.ds(0, C)],
            out_hbm.at[0, :, 0], ss[b]).wait()


def kernel(x, constant, weight):
    xt = _remap(x.astype(jnp.int32)).T.reshape(NJ, NW, C).transpose(1, 0, 2)
    ct = _remap(constant.astype(jnp.int32)).T.reshape(NJ, NW, C).transpose(1, 0, 2)
    res = _emb_add(xt, ct, _packed_table(weight))
    return res.transpose(2, 4, 0, 1, 3).reshape(NI, NJ, D)
